# baseline (speedup 1.0000x reference)
import jax
import jax.numpy as jnp
from jax.experimental import pallas as pl
from jax.experimental.pallas import tpu as pltpu

_N = 4096
_MAXDEG = 64
_BR = 256


def _row_scale_kernel(rank_ref, c_ref, mask_ref, out_ref):
    i = pl.program_id(0)
    r = rank_ref[0, pl.ds(i * _BR, _BR)]
    rc = jnp.minimum(r, _MAXDEG - 1)
    g = jnp.full((_BR,), c_ref[0], dtype=jnp.float32)
    for k in range(1, _MAXDEG):
        g = jnp.where(rc == k, c_ref[k], g)
    out_ref[...] = g[:, None] * mask_ref[...]


def kernel(x, rank, sparse_mask, c):
    del x
    grid = _N // _BR
    return pl.pallas_call(
        _row_scale_kernel,
        grid=(grid,),
        in_specs=[
            pl.BlockSpec((1, _N), lambda i: (0, 0)),
            pl.BlockSpec(memory_space=pltpu.SMEM),
            pl.BlockSpec((_BR, _N), lambda i: (i, 0)),
        ],
        out_specs=pl.BlockSpec((_BR, _N), lambda i: (i, 0)),
        out_shape=jax.ShapeDtypeStruct((_N, _N), jnp.float32),
    )(rank.reshape(1, _N), c, sparse_mask)


# speedup vs baseline: 1.0415x; 1.0415x over previous
import jax
import jax.numpy as jnp
from jax.experimental import pallas as pl
from jax.experimental.pallas import tpu as pltpu

_N = 4096
_MAXDEG = 64
_BR = 512


def _row_scale_kernel(rank_ref, c_ref, mask_ref, out_ref):
    i = pl.program_id(0)
    r = rank_ref[pl.ds(i * _BR, _BR)]
    rc = jnp.minimum(r, _MAXDEG - 1)
    g = jnp.full((_BR,), c_ref[0], dtype=jnp.float32)
    for k in range(1, _MAXDEG):
        g = jnp.where(rc == k, c_ref[k], g)
    out_ref[...] = g[:, None] * mask_ref[...]


def kernel(x, rank, sparse_mask, c):
    del x
    grid = _N // _BR
    return pl.pallas_call(
        _row_scale_kernel,
        grid=(grid,),
        in_specs=[
            pl.BlockSpec((_N,), lambda i: (0,)),
            pl.BlockSpec(memory_space=pltpu.SMEM),
            pl.BlockSpec((_BR, _N), lambda i: (i, 0)),
        ],
        out_specs=pl.BlockSpec((_BR, _N), lambda i: (i, 0)),
        out_shape=jax.ShapeDtypeStruct((_N, _N), jnp.float32),
    )(rank, c, sparse_mask)
